# baseline (speedup 1.0000x reference)
import jax
import jax.numpy as jnp
from jax import lax
from jax.experimental import pallas as pl
from jax.experimental.pallas import tpu as pltpu
from jax.experimental.pallas import tpu_sc as plsc

N = 10000
E = 320000
D = 128
DH = D // 2
NC = 2
NS = 16
NW = NC * NS
K = 128
NCHUNK = 80
EPWP = NCHUNK * K
EPAD = NW * EPWP
NTRASH = 256
NA = N + NTRASH
NBUF = 4
SB = 624
LAST_BASE = (NS - 1) * SB
LAST_SIZE = N - LAST_BASE
DEGW = 16

_MESH = plsc.VectorSubcoreMesh(core_axis_name="c", subcore_axis_name="s")
_SC_PARAMS = pltpu.CompilerParams(use_tc_tiling_on_sc=False)


def _deg_body(col_hbm, deg_hbm, col_v, ones_v, zbuf_v, acc_sh, sem):
    c = lax.axis_index("c")
    s = lax.axis_index("s")
    wid = s * NC + c

    def fill_ones(i, carry):
        ones_v[i, :] = jnp.full((DEGW,), 1.0, jnp.float32)
        return carry

    lax.fori_loop(0, K, fill_ones, 0)

    def fill_zero(i, carry):
        zbuf_v[i, :] = jnp.zeros((DEGW,), jnp.float32)
        return carry

    lax.fori_loop(0, LAST_SIZE + NTRASH, fill_zero, 0)

    base = pl.multiple_of(s * SB, 8)

    @pl.when(s < NS - 1)
    def _():
        pltpu.sync_copy(zbuf_v.at[pl.ds(0, SB)], acc_sh.at[pl.ds(base, SB)])

    @pl.when(s == NS - 1)
    def _():
        pltpu.sync_copy(zbuf_v,
                        acc_sh.at[pl.ds(LAST_BASE, LAST_SIZE + NTRASH)])

    pltpu.sync_copy(col_hbm.at[wid], col_v)
    plsc.subcore_barrier()

    def group(g, carry):
        for b in range(8):
            pltpu.async_copy(ones_v, acc_sh.at[col_v.at[g * 8 + b]], sem,
                             add=True)
        for b in range(8):
            pltpu.make_async_copy(ones_v, acc_sh.at[col_v.at[g * 8 + b]],
                                  sem).wait()
        return carry

    lax.fori_loop(0, NCHUNK // 8, group, 0)
    plsc.subcore_barrier()

    @pl.when(s < NS - 1)
    def _():
        pltpu.sync_copy(acc_sh.at[pl.ds(base, SB)],
                        deg_hbm.at[c, pl.ds(base, SB)])

    @pl.when(s == NS - 1)
    def _():
        pltpu.sync_copy(acc_sh.at[pl.ds(LAST_BASE, LAST_SIZE)],
                        deg_hbm.at[c, pl.ds(LAST_BASE, LAST_SIZE)])


_deg_call = pl.kernel(
    _deg_body,
    out_type=jax.ShapeDtypeStruct((NC, N, DEGW), jnp.float32),
    mesh=_MESH,
    compiler_params=_SC_PARAMS,
    scratch_types=[
        pltpu.VMEM((NCHUNK, K), jnp.int32),
        pltpu.VMEM((K, DEGW), jnp.float32),
        pltpu.VMEM((LAST_SIZE + NTRASH, DEGW), jnp.float32),
        pltpu.VMEM_SHARED((NA, DEGW), jnp.float32),
        pltpu.SemaphoreType.DMA,
    ],
)


def _agg_body(hsa_hbm, hsb_hbm, row_hbm, col_hbm, acc_hbm,
              row_v, col_v, buf_v, acc_sh, gsem):
    c = lax.axis_index("c")
    s = lax.axis_index("s")
    wid = s * NC + c
    base = pl.multiple_of(s * SB, 8)

    pltpu.sync_copy(row_hbm.at[wid], row_v)
    pltpu.sync_copy(col_hbm.at[wid], col_v)

    def gather(j, b, hs_hbm):
        pltpu.async_copy(hs_hbm.at[row_v.at[j]], buf_v.at[b], gsem.at[b])

    def gather_wait(j, b, hs_hbm):
        pltpu.make_async_copy(hs_hbm.at[row_v.at[j]], buf_v.at[b],
                              gsem.at[b]).wait()

    def sync_scatter(j, b):
        pltpu.sync_copy(buf_v.at[b], acc_sh.at[col_v.at[j]], add=True)

    for h, hs_hbm in ((0, hsa_hbm), (1, hsb_hbm)):
        @pl.when(s < NS - 1)
        def _():
            pltpu.sync_copy(hs_hbm.at[pl.ds(base, SB)],
                            acc_sh.at[pl.ds(base, SB)])

        @pl.when(s == NS - 1)
        def _():
            pltpu.sync_copy(hs_hbm.at[pl.ds(LAST_BASE, LAST_SIZE)],
                            acc_sh.at[pl.ds(LAST_BASE, LAST_SIZE)])

        plsc.subcore_barrier()

        for b in range(4):
            gather(b, b, hs_hbm)

        def group(g, carry):
            for b in range(4):
                j = g * 4 + b
                gather_wait(j, b, hs_hbm)
                sync_scatter(j, b)
                gather(j + 4, b, hs_hbm)
            return carry

        lax.fori_loop(0, NCHUNK // 4 - 1, group, 0)

        for b in range(4):
            j = NCHUNK - 4 + b
            gather_wait(j, b, hs_hbm)
            sync_scatter(j, b)

        plsc.subcore_barrier()

        @pl.when(s < NS - 1)
        def _():
            pltpu.sync_copy(acc_sh.at[pl.ds(base, SB)],
                            acc_hbm.at[c, pl.ds(base, SB),
                                       pl.ds(h * DH, DH)])

        @pl.when(s == NS - 1)
        def _():
            pltpu.sync_copy(acc_sh.at[pl.ds(LAST_BASE, LAST_SIZE)],
                            acc_hbm.at[c, pl.ds(LAST_BASE, LAST_SIZE),
                                       pl.ds(h * DH, DH)])

        if h == 0:
            plsc.subcore_barrier()


_agg_call = pl.kernel(
    _agg_body,
    out_type=jax.ShapeDtypeStruct((NC, N, D), jnp.float32),
    mesh=_MESH,
    compiler_params=_SC_PARAMS,
    scratch_types=[
        pltpu.VMEM((NCHUNK, K), jnp.int32),
        pltpu.VMEM((NCHUNK, K), jnp.int32),
        pltpu.VMEM((NBUF, K, DH), jnp.float32),
        pltpu.VMEM_SHARED((NA, DH), jnp.float32),
        pltpu.SemaphoreType.DMA((NBUF,)),
    ],
)


BR = 2000


def _dis_from(dp_ref):
    deg = dp_ref[0, :, 0] + dp_ref[1, :, 0] + 1.0
    return lax.rsqrt(deg)


def _lin1_body(dp_ref, x_ref, w_ref, oa_ref, ob_ref):
    dis = _dis_from(dp_ref)
    res = jnp.dot(x_ref[...], w_ref[...],
                  preferred_element_type=jnp.float32) * dis[:, None]
    oa_ref[...] = res[:, :DH]
    ob_ref[...] = res[:, DH:]


def _lin2_body(dp_ref, a0_ref, a1_ref, ha_ref, hb_ref, b_ref,
               w_ref, oa_ref, ob_ref):
    dis = _dis_from(dp_ref)
    agg = (a0_ref[...].reshape(BR, D) + a1_ref[...].reshape(BR, D)
           - jnp.concatenate([ha_ref[...], hb_ref[...]], axis=1))
    h = jnp.maximum(agg * dis[:, None] + b_ref[...], 0.0)
    res = jnp.dot(h, w_ref[...],
                  preferred_element_type=jnp.float32) * dis[:, None]
    oa_ref[...] = res[:, :DH]
    ob_ref[...] = res[:, DH:]


def _fin_body(dp_ref, a0_ref, a1_ref, ha_ref, hb_ref, b_ref,
              o_ref):
    dis = _dis_from(dp_ref)
    agg = (a0_ref[...].reshape(BR, D) + a1_ref[...].reshape(BR, D)
           - jnp.concatenate([ha_ref[...], hb_ref[...]], axis=1))
    o_ref[...] = agg * dis[:, None] + b_ref[...]


_dp_spec = pl.BlockSpec((NC, BR, DEGW), lambda i: (0, i, 0))
_row_spec = pl.BlockSpec((BR, D), lambda i: (i, 0))
_half_spec = pl.BlockSpec((BR, DH), lambda i: (i, 0))
_a0_spec = pl.BlockSpec((BR * D,), lambda i: (i,))
_a1_spec = pl.BlockSpec((BR * D,), lambda i: (i + N // BR,))
_w_spec = pl.BlockSpec((D, D), lambda i: (0, 0))
_b_spec = pl.BlockSpec((1, D), lambda i: (0, 0))
_half_sds = jax.ShapeDtypeStruct((N, DH), jnp.float32)
_full_sds = jax.ShapeDtypeStruct((N, D), jnp.float32)

_lin1_call = pl.pallas_call(
    _lin1_body, grid=(N // BR,),
    in_specs=[_dp_spec, _row_spec, _w_spec],
    out_specs=[_half_spec, _half_spec], out_shape=[_half_sds, _half_sds])

_lin2_call = pl.pallas_call(
    _lin2_body, grid=(N // BR,),
    in_specs=[_dp_spec, _a0_spec, _a1_spec, _half_spec,
              _half_spec, _b_spec, _w_spec],
    out_specs=[_half_spec, _half_spec], out_shape=[_half_sds, _half_sds])

_fin_call = pl.pallas_call(
    _fin_body, grid=(N // BR,),
    in_specs=[_dp_spec, _a0_spec, _a1_spec, _half_spec,
              _half_spec, _b_spec],
    out_specs=_row_spec, out_shape=_full_sds)


def kernel(x, edge_index, W1, b1, W2, b2):
    ei = edge_index.astype(jnp.int32)
    pad = EPWP - E // NW
    row2 = ei[0].reshape(NW, E // NW)
    col2 = ei[1].reshape(NW, E // NW)
    pidx = jnp.arange(NW * pad, dtype=jnp.int32).reshape(NW, pad)
    prow = pidx % N
    pcol = N + (pidx % NTRASH)
    row = jnp.concatenate([row2, prow], axis=1).reshape(NW, NCHUNK, K)
    col = jnp.concatenate([col2, pcol], axis=1).reshape(NW, NCHUNK, K)
    b1r = b1.reshape(1, D)
    b2r = b2.reshape(1, D)

    degf = _deg_call(col)
    h1a, h1b = _lin1_call(degf, x, W1)
    a1 = _agg_call(h1a, h1b, row, col)
    a1f = a1.reshape(-1)
    h2a, h2b = _lin2_call(degf, a1f, a1f, h1a, h1b, b1r, W2)
    a2 = _agg_call(h2a, h2b, row, col)
    a2f = a2.reshape(-1)
    out = _fin_call(degf, a2f, a2f, h2a, h2b, b2r)
    return out


# speedup vs baseline: 1.0022x; 1.0022x over previous
import jax
import jax.numpy as jnp
from jax import lax
from jax.experimental import pallas as pl
from jax.experimental.pallas import tpu as pltpu
from jax.experimental.pallas import tpu_sc as plsc

N = 10000
E = 320000
D = 128
DH = D // 2
NC = 2
NS = 16
NW = NC * NS
K = 128
NCHUNK = 80
EPWP = NCHUNK * K
EPAD = NW * EPWP
NTRASH = 256
NA = N + NTRASH
NBUF = 4
SB = 624
LAST_BASE = (NS - 1) * SB
LAST_SIZE = N - LAST_BASE
DEGW = 16

_MESH = plsc.VectorSubcoreMesh(core_axis_name="c", subcore_axis_name="s")
_SC_PARAMS = pltpu.CompilerParams(use_tc_tiling_on_sc=False)


def _deg_body(col_hbm, deg_hbm, col_v, ones_v, zbuf_v, acc_sh, sem):
    c = lax.axis_index("c")
    s = lax.axis_index("s")
    wid = s * NC + c

    def fill_ones(i, carry):
        ones_v[i, :] = jnp.full((DEGW,), 1.0, jnp.float32)
        return carry

    lax.fori_loop(0, K, fill_ones, 0)

    def fill_zero(i, carry):
        zbuf_v[i, :] = jnp.zeros((DEGW,), jnp.float32)
        return carry

    lax.fori_loop(0, LAST_SIZE + NTRASH, fill_zero, 0)

    base = pl.multiple_of(s * SB, 8)

    @pl.when(s < NS - 1)
    def _():
        pltpu.sync_copy(zbuf_v.at[pl.ds(0, SB)], acc_sh.at[pl.ds(base, SB)])

    @pl.when(s == NS - 1)
    def _():
        pltpu.sync_copy(zbuf_v,
                        acc_sh.at[pl.ds(LAST_BASE, LAST_SIZE + NTRASH)])

    pltpu.sync_copy(col_hbm.at[wid], col_v)
    plsc.subcore_barrier()

    def group(g, carry):
        for b in range(8):
            pltpu.async_copy(ones_v, acc_sh.at[col_v.at[g * 8 + b]], sem,
                             add=True)
        for b in range(8):
            pltpu.make_async_copy(ones_v, acc_sh.at[col_v.at[g * 8 + b]],
                                  sem).wait()
        return carry

    lax.fori_loop(0, NCHUNK // 8, group, 0)
    plsc.subcore_barrier()

    @pl.when(s < NS - 1)
    def _():
        pltpu.sync_copy(acc_sh.at[pl.ds(base, SB)],
                        deg_hbm.at[c, pl.ds(base, SB)])

    @pl.when(s == NS - 1)
    def _():
        pltpu.sync_copy(acc_sh.at[pl.ds(LAST_BASE, LAST_SIZE)],
                        deg_hbm.at[c, pl.ds(LAST_BASE, LAST_SIZE)])


_deg_call = pl.kernel(
    _deg_body,
    out_type=jax.ShapeDtypeStruct((NC, N, DEGW), jnp.float32),
    mesh=_MESH,
    compiler_params=_SC_PARAMS,
    scratch_types=[
        pltpu.VMEM((NCHUNK, K), jnp.int32),
        pltpu.VMEM((K, DEGW), jnp.float32),
        pltpu.VMEM((LAST_SIZE + NTRASH, DEGW), jnp.float32),
        pltpu.VMEM_SHARED((NA, DEGW), jnp.float32),
        pltpu.SemaphoreType.DMA,
    ],
)


def _agg_body(hsa_hbm, hsb_hbm, row_hbm, col_hbm, acc_hbm,
              row_v, col_v, buf_v, acc_sh, gsem, ssem):
    c = lax.axis_index("c")
    s = lax.axis_index("s")
    wid = s * NC + c
    base = pl.multiple_of(s * SB, 8)

    pltpu.sync_copy(row_hbm.at[wid], row_v)
    pltpu.sync_copy(col_hbm.at[wid], col_v)

    def gather(j, b, hs_hbm):
        pltpu.async_copy(hs_hbm.at[row_v.at[j]], buf_v.at[b], gsem.at[b])

    def gather_wait(j, b, hs_hbm):
        pltpu.make_async_copy(hs_hbm.at[row_v.at[j]], buf_v.at[b],
                              gsem.at[b]).wait()

    def scatter(j, b):
        pltpu.async_copy(buf_v.at[b], acc_sh.at[col_v.at[j]], ssem,
                         add=True)

    def scatter_wait(j, b):
        pltpu.make_async_copy(buf_v.at[b], acc_sh.at[col_v.at[j]],
                              ssem).wait()

    for h, hs_hbm in ((0, hsa_hbm), (1, hsb_hbm)):
        @pl.when(s < NS - 1)
        def _():
            pltpu.sync_copy(hs_hbm.at[pl.ds(base, SB)],
                            acc_sh.at[pl.ds(base, SB)])

        @pl.when(s == NS - 1)
        def _():
            pltpu.sync_copy(hs_hbm.at[pl.ds(LAST_BASE, LAST_SIZE)],
                            acc_sh.at[pl.ds(LAST_BASE, LAST_SIZE)])

        plsc.subcore_barrier()

        for b in range(3):
            gather(b, b, hs_hbm)
        gather_wait(0, 0, hs_hbm)
        gather(3, 3, hs_hbm)
        scatter(0, 0)
        for b in range(1, 4):
            gather_wait(b, b, hs_hbm)
            scatter_wait(b - 1, (b - 1) % 4)
            gather(b + 3, (b + 3) % 4, hs_hbm)
            scatter(b, b)

        def group(g, carry):
            for b in range(4):
                j = g * 4 + b
                gather_wait(j, b, hs_hbm)
                scatter_wait(j - 1, (b + 3) % 4)
                gather(j + 3, (b + 3) % 4, hs_hbm)
                scatter(j, b)
            return carry

        lax.fori_loop(1, NCHUNK // 4 - 1, group, 0)

        for b in range(4):
            j = NCHUNK - 4 + b
            gather_wait(j, b, hs_hbm)
            scatter_wait(j - 1, (b + 3) % 4)
            if j + 3 < NCHUNK:
                gather(j + 3, (b + 3) % 4, hs_hbm)
            scatter(j, b)
        scatter_wait(NCHUNK - 1, 3)

        plsc.subcore_barrier()

        @pl.when(s < NS - 1)
        def _():
            pltpu.sync_copy(acc_sh.at[pl.ds(base, SB)],
                            acc_hbm.at[c, pl.ds(base, SB),
                                       pl.ds(h * DH, DH)])

        @pl.when(s == NS - 1)
        def _():
            pltpu.sync_copy(acc_sh.at[pl.ds(LAST_BASE, LAST_SIZE)],
                            acc_hbm.at[c, pl.ds(LAST_BASE, LAST_SIZE),
                                       pl.ds(h * DH, DH)])

        if h == 0:
            plsc.subcore_barrier()


_agg_call = pl.kernel(
    _agg_body,
    out_type=jax.ShapeDtypeStruct((NC, N, D), jnp.float32),
    mesh=_MESH,
    compiler_params=_SC_PARAMS,
    scratch_types=[
        pltpu.VMEM((NCHUNK, K), jnp.int32),
        pltpu.VMEM((NCHUNK, K), jnp.int32),
        pltpu.VMEM((NBUF, K, DH), jnp.float32),
        pltpu.VMEM_SHARED((NA, DH), jnp.float32),
        pltpu.SemaphoreType.DMA((NBUF,)),
        pltpu.SemaphoreType.DMA,
    ],
)


BR = 2000


def _dis_from(dp_ref):
    deg = dp_ref[0, :, 0] + dp_ref[1, :, 0] + 1.0
    return lax.rsqrt(deg)


def _lin1_body(dp_ref, x_ref, w_ref, oa_ref, ob_ref):
    dis = _dis_from(dp_ref)
    res = jnp.dot(x_ref[...], w_ref[...],
                  preferred_element_type=jnp.float32) * dis[:, None]
    oa_ref[...] = res[:, :DH]
    ob_ref[...] = res[:, DH:]


def _lin2_body(dp_ref, acc_ref, ha_ref, hb_ref, b_ref, w_ref,
               oa_ref, ob_ref):
    dis = _dis_from(dp_ref)
    agg = (acc_ref[0] + acc_ref[1]
           - jnp.concatenate([ha_ref[...], hb_ref[...]], axis=1))
    h = jnp.maximum(agg * dis[:, None] + b_ref[...], 0.0)
    res = jnp.dot(h, w_ref[...],
                  preferred_element_type=jnp.float32) * dis[:, None]
    oa_ref[...] = res[:, :DH]
    ob_ref[...] = res[:, DH:]


def _fin_body(dp_ref, acc_ref, ha_ref, hb_ref, b_ref, o_ref):
    dis = _dis_from(dp_ref)
    agg = (acc_ref[0] + acc_ref[1]
           - jnp.concatenate([ha_ref[...], hb_ref[...]], axis=1))
    o_ref[...] = agg * dis[:, None] + b_ref[...]


_dp_spec = pl.BlockSpec((NC, BR, DEGW), lambda i: (0, i, 0))
_row_spec = pl.BlockSpec((BR, D), lambda i: (i, 0))
_half_spec = pl.BlockSpec((BR, DH), lambda i: (i, 0))
_acc_spec = pl.BlockSpec((NC, BR, D), lambda i: (0, i, 0))
_w_spec = pl.BlockSpec((D, D), lambda i: (0, 0))
_b_spec = pl.BlockSpec((1, D), lambda i: (0, 0))
_half_sds = jax.ShapeDtypeStruct((N, DH), jnp.float32)
_full_sds = jax.ShapeDtypeStruct((N, D), jnp.float32)

_lin1_call = pl.pallas_call(
    _lin1_body, grid=(N // BR,),
    in_specs=[_dp_spec, _row_spec, _w_spec],
    out_specs=[_half_spec, _half_spec], out_shape=[_half_sds, _half_sds])

_lin2_call = pl.pallas_call(
    _lin2_body, grid=(N // BR,),
    in_specs=[_dp_spec, _acc_spec, _half_spec, _half_spec, _b_spec, _w_spec],
    out_specs=[_half_spec, _half_spec], out_shape=[_half_sds, _half_sds])

_fin_call = pl.pallas_call(
    _fin_body, grid=(N // BR,),
    in_specs=[_dp_spec, _acc_spec, _half_spec, _half_spec, _b_spec],
    out_specs=_row_spec, out_shape=_full_sds)


def kernel(x, edge_index, W1, b1, W2, b2):
    ei = edge_index.astype(jnp.int32)
    pad = EPWP - E // NW
    row2 = ei[0].reshape(NW, E // NW)
    col2 = ei[1].reshape(NW, E // NW)
    pidx = jnp.arange(NW * pad, dtype=jnp.int32).reshape(NW, pad)
    prow = pidx % N
    pcol = N + (pidx % NTRASH)
    row = jnp.concatenate([row2, prow], axis=1).reshape(NW, NCHUNK, K)
    col = jnp.concatenate([col2, pcol], axis=1).reshape(NW, NCHUNK, K)
    b1r = b1.reshape(1, D)
    b2r = b2.reshape(1, D)

    deg_parts = _deg_call(col)
    h1a, h1b = _lin1_call(deg_parts, x, W1)
    a1 = _agg_call(h1a, h1b, row, col)
    h2a, h2b = _lin2_call(deg_parts, a1, h1a, h1b, b1r, W2)
    a2 = _agg_call(h2a, h2b, row, col)
    out = _fin_call(deg_parts, a2, h2a, h2b, b2r)
    return out
